# speedup vs baseline: 1.0142x; 1.0142x over previous
import jax
import jax.numpy as jnp
from jax import lax
from jax.experimental import pallas as pl
from jax.experimental.pallas import tpu as pltpu
from jax.experimental.pallas import tpu_sc as plsc

VOCAB = 10000
DIM = 512
SEQ = 2048
BATCH = 64
N = SEQ * BATCH

NC = 2
NS = 16
NW = NC * NS
PER_W = N // NW
CHUNK = 64
N_CHUNKS = PER_W // CHUNK
LANES = 16
DV = DIM // LANES


def _embed_kernel(tok_hbm, pos_hbm, fpos_hbm, emb_hbm, post_hbm, fpost_hbm,
                  out_hbm, idx_t, idx_p, idx_f, rows_a, rows_b, sem):
    wid = lax.axis_index("s") * NC + lax.axis_index("c")
    w_base = wid * PER_W

    def chunk_body(i, carry):
        base = w_base + i * CHUNK
        pltpu.sync_copy(tok_hbm.at[pl.ds(base, CHUNK)], idx_t)
        pltpu.sync_copy(pos_hbm.at[pl.ds(base, CHUNK)], idx_p)
        pltpu.sync_copy(fpos_hbm.at[pl.ds(base, CHUNK)], idx_f)

        pltpu.async_copy(emb_hbm.at[idx_t], rows_a, sem).wait()
        pltpu.async_copy(post_hbm.at[idx_p], rows_b, sem).wait()

        def add_row(r, c):
            for j in range(DV):
                sl = (r, pl.ds(j * LANES, LANES))
                rows_a[sl] = rows_a[sl] + rows_b[sl]
            return c

        lax.fori_loop(0, CHUNK, add_row, 0)

        pltpu.async_copy(fpost_hbm.at[idx_f], rows_b, sem).wait()
        lax.fori_loop(0, CHUNK, add_row, 0)

        pltpu.sync_copy(rows_a, out_hbm.at[pl.ds(base, CHUNK)])
        return carry

    lax.fori_loop(0, N_CHUNKS, chunk_body, 0)


@jax.jit
def _run(tok, pos, fpos, emb_table, pos_table, fpos_table):
    mesh = plsc.VectorSubcoreMesh(core_axis_name="c", subcore_axis_name="s")
    call = pl.kernel(
        _embed_kernel,
        mesh=mesh,
        out_type=jax.ShapeDtypeStruct((N, DIM), jnp.float32),
        scratch_types=[
            pltpu.VMEM((CHUNK,), jnp.int32),
            pltpu.VMEM((CHUNK,), jnp.int32),
            pltpu.VMEM((CHUNK,), jnp.int32),
            pltpu.VMEM((CHUNK, DIM), jnp.float32),
            pltpu.VMEM((CHUNK, DIM), jnp.float32),
            pltpu.SemaphoreType.DMA,
        ],
    )
    out = call(tok, pos, fpos, emb_table, pos_table, fpos_table)
    return out.reshape(SEQ, BATCH, DIM)


def kernel(batch_datasets, batch_positionals, batch_float_positionals,
           emb_table, pos_table, fpos_table):
    tok = batch_datasets.reshape(N).astype(jnp.int32)
    pos = batch_positionals.reshape(N).astype(jnp.int32)
    fpos = batch_float_positionals.reshape(N).astype(jnp.int32)
    return _run(tok, pos, fpos, emb_table, pos_table, fpos_table)


# speedup vs baseline: 2.4323x; 2.3983x over previous
import jax
import jax.numpy as jnp
from jax import lax
from jax.experimental import pallas as pl
from jax.experimental.pallas import tpu as pltpu
from jax.experimental.pallas import tpu_sc as plsc

VOCAB = 10000
DIM = 512
SEQ = 2048
BATCH = 64
N = SEQ * BATCH
N_POS = 13
N_FPOS = 5
N_COMBO = N_POS * N_FPOS

NC = 2
NS = 16
NW = NC * NS
PER_W = N // NW
CHUNK = 64
N_CHUNKS = PER_W // CHUNK
LANES = 16
DV = DIM // LANES


def _embed_kernel(tok_hbm, pos_hbm, fpos_hbm, emb_hbm, post_hbm, fpost_hbm,
                  out_hbm, idx_t, idx_p, idx_f, idx_c, rows, pos_v, fpos_v,
                  combo_v, sem_g0, sem_g1, sem_o0, sem_o1, sem_i):
    sem_g = [sem_g0, sem_g1]
    sem_o = [sem_o0, sem_o1]
    wid = lax.axis_index("s") * NC + lax.axis_index("c")
    w_base = wid * PER_W

    pltpu.sync_copy(post_hbm, pos_v)
    pltpu.sync_copy(fpost_hbm, fpos_v)

    def build_combo(c, carry):
        i = c // N_FPOS
        j = c - i * N_FPOS
        for k in range(DV):
            sl = pl.ds(k * LANES, LANES)
            combo_v[c, sl] = pos_v[i, sl] + fpos_v[j, sl]
        return carry

    lax.fori_loop(0, N_COMBO, build_combo, 0)

    def stage_idx(ci, b):
        base = w_base + ci * CHUNK
        h1 = pltpu.async_copy(tok_hbm.at[pl.ds(base, CHUNK)], idx_t.at[b], sem_i)
        h2 = pltpu.async_copy(pos_hbm.at[pl.ds(base, CHUNK)], idx_p.at[b], sem_i)
        h3 = pltpu.async_copy(fpos_hbm.at[pl.ds(base, CHUNK)], idx_f.at[b], sem_i)
        h1.wait()
        h2.wait()
        h3.wait()
        for k in range(CHUNK // LANES):
            sl = pl.ds(k * LANES, LANES)
            idx_c[b, sl] = idx_p[b, sl] * N_FPOS + idx_f[b, sl]

    def issue_gather(b):
        pltpu.async_copy(emb_hbm.at[idx_t.at[b]], rows.at[b], sem_g[b])

    def wait_gather(b):
        pltpu.make_async_copy(emb_hbm.at[pl.ds(0, CHUNK)], rows.at[b],
                              sem_g[b]).wait()

    def issue_scatter(ci, b):
        base = w_base + ci * CHUNK
        pltpu.async_copy(rows.at[b], out_hbm.at[pl.ds(base, CHUNK)], sem_o[b])

    def wait_scatter(b):
        pltpu.make_async_copy(rows.at[b], out_hbm.at[pl.ds(0, CHUNK)],
                              sem_o[b]).wait()

    def add_pass(b):
        def add_row(r, carry):
            c = idx_c[b, pl.ds(r, LANES)][0]
            for k in range(DV):
                sl = pl.ds(k * LANES, LANES)
                rows[b, r, sl] = rows[b, r, sl] + combo_v[c, sl]
            return carry

        lax.fori_loop(0, CHUNK, add_row, 0)

    stage_idx(0, 0)
    issue_gather(0)

    def outer(go, carry):
        for b in range(2):
            i = go * 2 + b
            nb = 1 - b

            @pl.when(i + 1 < N_CHUNKS)
            def _stage():
                stage_idx(i + 1, nb)

                @pl.when(i >= 1)
                def _drain():
                    wait_scatter(nb)

                issue_gather(nb)

            wait_gather(b)
            add_pass(b)
            issue_scatter(i, b)
        return carry

    lax.fori_loop(0, N_CHUNKS // 2, outer, 0)
    wait_scatter(0)
    wait_scatter(1)


@jax.jit
def _run(tok, pos, fpos, emb_table, pos_table, fpos_table):
    mesh = plsc.VectorSubcoreMesh(core_axis_name="c", subcore_axis_name="s")
    call = pl.kernel(
        _embed_kernel,
        mesh=mesh,
        out_type=jax.ShapeDtypeStruct((N, DIM), jnp.float32),
        scratch_types=[
            pltpu.VMEM((2, CHUNK), jnp.int32),
            pltpu.VMEM((2, CHUNK), jnp.int32),
            pltpu.VMEM((2, CHUNK), jnp.int32),
            pltpu.VMEM((2, CHUNK + LANES), jnp.int32),
            pltpu.VMEM((2, CHUNK, DIM), jnp.float32),
            pltpu.VMEM((N_POS, DIM), jnp.float32),
            pltpu.VMEM((N_FPOS, DIM), jnp.float32),
            pltpu.VMEM((N_COMBO, DIM), jnp.float32),
            pltpu.SemaphoreType.DMA,
            pltpu.SemaphoreType.DMA,
            pltpu.SemaphoreType.DMA,
            pltpu.SemaphoreType.DMA,
            pltpu.SemaphoreType.DMA,
        ],
    )
    out = call(tok, pos, fpos, emb_table, pos_table, fpos_table)
    return out.reshape(SEQ, BATCH, DIM)


def kernel(batch_datasets, batch_positionals, batch_float_positionals,
           emb_table, pos_table, fpos_table):
    tok = batch_datasets.reshape(N).astype(jnp.int32)
    pos = batch_positionals.reshape(N).astype(jnp.int32)
    fpos = batch_float_positionals.reshape(N).astype(jnp.int32)
    return _run(tok, pos, fpos, emb_table, pos_table, fpos_table)


# speedup vs baseline: 6.6406x; 2.7302x over previous
import jax
import jax.numpy as jnp
from jax import lax
from jax.experimental import pallas as pl
from jax.experimental.pallas import tpu as pltpu
from jax.experimental.pallas import tpu_sc as plsc

VOCAB = 10000
DIM = 512
SEQ = 2048
BATCH = 64
N = SEQ * BATCH
N_POS = 13
N_FPOS = 5
N_COMBO = N_POS * N_FPOS

NC = 2
NS = 16
NW = NC * NS
PER_W = N // NW
CHUNK = 64
N_CHUNKS = PER_W // CHUNK
LANES = 16
DV = DIM // LANES


def _embed_kernel(tok_hbm, pos_hbm, fpos_hbm, emb_hbm, post_hbm, fpost_hbm,
                  out_hbm, idx_t, idx_p, idx_f, idx_c, rows, pos_v, fpos_v,
                  combo_v, sem_g0, sem_g1, sem_o0, sem_o1, sem_i):
    sem_g = [sem_g0, sem_g1]
    sem_o = [sem_o0, sem_o1]
    wid = lax.axis_index("s") * NC + lax.axis_index("c")
    w_base = wid * PER_W

    pltpu.sync_copy(post_hbm, pos_v)
    pltpu.sync_copy(fpost_hbm, fpos_v)

    @plsc.parallel_loop(0, N_COMBO, 1, unroll=2)
    def build_combo(c):
        i = c // N_FPOS
        j = c - i * N_FPOS
        for k in range(DV):
            sl = pl.ds(k * LANES, LANES)
            combo_v[c, sl] = pos_v[i, sl] + fpos_v[j, sl]

    def stage_idx(ci, b):
        base = w_base + ci * CHUNK
        h1 = pltpu.async_copy(tok_hbm.at[pl.ds(base, CHUNK)], idx_t.at[b], sem_i)
        h2 = pltpu.async_copy(pos_hbm.at[pl.ds(base, CHUNK)], idx_p.at[b], sem_i)
        h3 = pltpu.async_copy(fpos_hbm.at[pl.ds(base, CHUNK)], idx_f.at[b], sem_i)
        h1.wait()
        h2.wait()
        h3.wait()
        for k in range(CHUNK // LANES):
            sl = pl.ds(k * LANES, LANES)
            idx_c[b, sl] = idx_p[b, sl] * N_FPOS + idx_f[b, sl]

    def issue_gather(b):
        pltpu.async_copy(emb_hbm.at[idx_t.at[b]], rows.at[b], sem_g[b])

    def wait_gather(b):
        pltpu.make_async_copy(emb_hbm.at[pl.ds(0, CHUNK)], rows.at[b],
                              sem_g[b]).wait()

    def issue_scatter(ci, b):
        base = w_base + ci * CHUNK
        pltpu.async_copy(rows.at[b], out_hbm.at[pl.ds(base, CHUNK)], sem_o[b])

    def wait_scatter(b):
        pltpu.make_async_copy(rows.at[b], out_hbm.at[pl.ds(0, CHUNK)],
                              sem_o[b]).wait()

    def add_pass(b):
        @plsc.parallel_loop(0, CHUNK, 1, unroll=2)
        def add_row(r):
            c = idx_c[b, pl.ds(r, LANES)][0]
            for k in range(DV):
                sl = pl.ds(k * LANES, LANES)
                rows[b, r, sl] = rows[b, r, sl] + combo_v[c, sl]

    stage_idx(0, 0)
    issue_gather(0)

    def outer(go, carry):
        for b in range(2):
            i = go * 2 + b
            nb = 1 - b

            @pl.when(i + 1 < N_CHUNKS)
            def _stage():
                stage_idx(i + 1, nb)

                @pl.when(i >= 1)
                def _drain():
                    wait_scatter(nb)

                issue_gather(nb)

            wait_gather(b)
            add_pass(b)
            issue_scatter(i, b)
        return carry

    lax.fori_loop(0, N_CHUNKS // 2, outer, 0)
    wait_scatter(0)
    wait_scatter(1)


@jax.jit
def _run(tok, pos, fpos, emb_table, pos_table, fpos_table):
    mesh = plsc.VectorSubcoreMesh(core_axis_name="c", subcore_axis_name="s")
    call = pl.kernel(
        _embed_kernel,
        mesh=mesh,
        out_type=jax.ShapeDtypeStruct((N, DIM), jnp.float32),
        scratch_types=[
            pltpu.VMEM((2, CHUNK), jnp.int32),
            pltpu.VMEM((2, CHUNK), jnp.int32),
            pltpu.VMEM((2, CHUNK), jnp.int32),
            pltpu.VMEM((2, CHUNK + LANES), jnp.int32),
            pltpu.VMEM((2, CHUNK, DIM), jnp.float32),
            pltpu.VMEM((N_POS, DIM), jnp.float32),
            pltpu.VMEM((N_FPOS, DIM), jnp.float32),
            pltpu.VMEM((N_COMBO, DIM), jnp.float32),
            pltpu.SemaphoreType.DMA,
            pltpu.SemaphoreType.DMA,
            pltpu.SemaphoreType.DMA,
            pltpu.SemaphoreType.DMA,
            pltpu.SemaphoreType.DMA,
        ],
    )
    out = call(tok, pos, fpos, emb_table, pos_table, fpos_table)
    return out.reshape(SEQ, BATCH, DIM)


def kernel(batch_datasets, batch_positionals, batch_float_positionals,
           emb_table, pos_table, fpos_table):
    tok = batch_datasets.reshape(N).astype(jnp.int32)
    pos = batch_positionals.reshape(N).astype(jnp.int32)
    fpos = batch_float_positionals.reshape(N).astype(jnp.int32)
    return _run(tok, pos, fpos, emb_table, pos_table, fpos_table)


# speedup vs baseline: 6.9289x; 1.0434x over previous
import jax
import jax.numpy as jnp
from jax import lax
from jax.experimental import pallas as pl
from jax.experimental.pallas import tpu as pltpu
from jax.experimental.pallas import tpu_sc as plsc

VOCAB = 10000
DIM = 512
SEQ = 2048
BATCH = 64
N = SEQ * BATCH
N_POS = 13
N_FPOS = 5
N_COMBO = N_POS * N_FPOS

NC = 2
NS = 16
NW = NC * NS
PER_W = N // NW
CHUNK = 64
N_CHUNKS = PER_W // CHUNK
LANES = 16
DV = DIM // LANES


def _embed_kernel(tok_hbm, pos_hbm, fpos_hbm, emb_hbm, post_hbm, fpost_hbm,
                  out_hbm, idx_t, idx_p, idx_f, idx_c, rows, pos_v, fpos_v,
                  combo_v, sem_g0, sem_g1, sem_o0, sem_o1, sem_i):
    sem_g = [sem_g0, sem_g1]
    sem_o = [sem_o0, sem_o1]
    wid = lax.axis_index("s") * NC + lax.axis_index("c")
    w_base = wid * PER_W

    pltpu.sync_copy(post_hbm, pos_v)
    pltpu.sync_copy(fpost_hbm, fpos_v)

    @plsc.parallel_loop(0, N_COMBO, 1, unroll=2)
    def build_combo(c):
        i = c // N_FPOS
        j = c - i * N_FPOS
        for k in range(DV):
            sl = pl.ds(k * LANES, LANES)
            combo_v[c, sl] = pos_v[i, sl] + fpos_v[j, sl]

    def stage_idx(ci, b):
        base = w_base + ci * CHUNK
        h1 = pltpu.async_copy(tok_hbm.at[pl.ds(base, CHUNK)], idx_t.at[b], sem_i)
        h2 = pltpu.async_copy(pos_hbm.at[pl.ds(base, CHUNK)], idx_p.at[b], sem_i)
        h3 = pltpu.async_copy(fpos_hbm.at[pl.ds(base, CHUNK)], idx_f.at[b], sem_i)
        h1.wait()
        h2.wait()
        h3.wait()
        for k in range(CHUNK // LANES):
            sl = pl.ds(k * LANES, LANES)
            idx_c[b, sl] = idx_p[b, sl] * N_FPOS + idx_f[b, sl]

    def issue_gather(b):
        pltpu.async_copy(emb_hbm.at[idx_t.at[b]], rows.at[b], sem_g[b])

    def wait_gather(b):
        pltpu.make_async_copy(emb_hbm.at[pl.ds(0, CHUNK)], rows.at[b],
                              sem_g[b]).wait()

    def issue_scatter(ci, b):
        base = w_base + ci * CHUNK
        pltpu.async_copy(rows.at[b], out_hbm.at[pl.ds(base, CHUNK)], sem_o[b])

    def wait_scatter(b):
        pltpu.make_async_copy(rows.at[b], out_hbm.at[pl.ds(0, CHUNK)],
                              sem_o[b]).wait()

    def add_pass(b):
        @plsc.parallel_loop(0, CHUNK, 1, unroll=2)
        def add_row(r):
            c = idx_c[b, pl.ds(r, LANES)][0]
            for k in range(DV):
                sl = pl.ds(k * LANES, LANES)
                plsc.addupdate(rows.at[b, r, sl], combo_v[c, sl])

    stage_idx(0, 0)
    issue_gather(0)

    def outer(go, carry):
        for b in range(2):
            i = go * 2 + b
            nb = 1 - b

            @pl.when(i + 1 < N_CHUNKS)
            def _stage():
                stage_idx(i + 1, nb)

                @pl.when(i >= 1)
                def _drain():
                    wait_scatter(nb)

                issue_gather(nb)

            wait_gather(b)
            add_pass(b)
            issue_scatter(i, b)
        return carry

    lax.fori_loop(0, N_CHUNKS // 2, outer, 0)
    wait_scatter(0)
    wait_scatter(1)


@jax.jit
def _run(tok, pos, fpos, emb_table, pos_table, fpos_table):
    mesh = plsc.VectorSubcoreMesh(core_axis_name="c", subcore_axis_name="s")
    call = pl.kernel(
        _embed_kernel,
        mesh=mesh,
        out_type=jax.ShapeDtypeStruct((N, DIM), jnp.float32),
        scratch_types=[
            pltpu.VMEM((2, CHUNK), jnp.int32),
            pltpu.VMEM((2, CHUNK), jnp.int32),
            pltpu.VMEM((2, CHUNK), jnp.int32),
            pltpu.VMEM((2, CHUNK + LANES), jnp.int32),
            pltpu.VMEM((2, CHUNK, DIM), jnp.float32),
            pltpu.VMEM((N_POS, DIM), jnp.float32),
            pltpu.VMEM((N_FPOS, DIM), jnp.float32),
            pltpu.VMEM((N_COMBO, DIM), jnp.float32),
            pltpu.SemaphoreType.DMA,
            pltpu.SemaphoreType.DMA,
            pltpu.SemaphoreType.DMA,
            pltpu.SemaphoreType.DMA,
            pltpu.SemaphoreType.DMA,
        ],
    )
    out = call(tok, pos, fpos, emb_table, pos_table, fpos_table)
    return out.reshape(SEQ, BATCH, DIM)


def kernel(batch_datasets, batch_positionals, batch_float_positionals,
           emb_table, pos_table, fpos_table):
    tok = batch_datasets.reshape(N).astype(jnp.int32)
    pos = batch_positionals.reshape(N).astype(jnp.int32)
    fpos = batch_float_positionals.reshape(N).astype(jnp.int32)
    return _run(tok, pos, fpos, emb_table, pos_table, fpos_table)
